# baseline (speedup 1.0000x reference)
import functools

import jax
import jax.numpy as jnp
from jax import lax
from jax.experimental import pallas as pl
from jax.experimental.pallas import tpu as pltpu

_DEPTH = 4


def _se_kernel(x_hbm, w1_ref, b1_ref, w2_ref, b2_ref, o_ref,
               bufs, sems, acc, *, n_hw, thw, inv_hw):
    def issue(k):
        slot = lax.rem(k, _DEPTH)
        pltpu.make_async_copy(
            x_hbm.at[pl.ds(k * thw, thw)], bufs.at[slot], sems.at[slot]
        ).start()

    for k in range(min(_DEPTH, n_hw)):
        issue(k)

    def body(k, _):
        slot = lax.rem(k, _DEPTH)
        pltpu.make_async_copy(bufs.at[slot], bufs.at[slot],
                              sems.at[slot]).wait()

        @pl.when(k + _DEPTH < n_hw)
        def _():
            issue(k + _DEPTH)

        s = jnp.sum(bufs[slot], axis=0)

        @pl.when(k == 0)
        def _():
            acc[...] = s

        @pl.when(k > 0)
        def _():
            acc[...] += s
        return ()

    lax.fori_loop(0, n_hw, body, (), unroll=False)

    p = acc[...] * inv_hw
    h = lax.dot_general(p, w1_ref[...], (((1,), (1,)), ((), ())),
                        preferred_element_type=jnp.float32)
    h = jnp.maximum(h + b1_ref[...], 0.0)
    y = lax.dot_general(h, w2_ref[...], (((1,), (1,)), ((), ())),
                        preferred_element_type=jnp.float32)
    o_ref[...] = jax.nn.sigmoid(y + b2_ref[...])


def _pick_thw(HW, B, C, esize, target_bytes):
    best = 1
    for d in range(1, HW + 1):
        if HW % d == 0 and d * B * C * esize <= target_bytes:
            best = d
    return best


def kernel(x_nchw, w1, b1, w2, b2):
    B, C, H, W = x_nchw.shape
    Cr = w1.shape[0]
    HW = H * W
    esize = jnp.dtype(x_nchw.dtype).itemsize

    xt = jnp.transpose(x_nchw, (2, 3, 0, 1)).reshape(HW, B, C)

    THW = _pick_thw(HW, B, C, esize, 7 << 20)
    n_hw = HW // THW

    b1r = b1.reshape(1, Cr).astype(jnp.float32)
    b2r = b2.reshape(1, C).astype(jnp.float32)

    block_bytes = THW * B * C * esize
    vmem_limit = int(min(_DEPTH * block_bytes + (8 << 20), 100 << 20))

    cost = pl.CostEstimate(
        flops=int(B * C * HW + 2 * B * C * Cr * 2),
        transcendentals=int(B * C),
        bytes_accessed=int(xt.size * esize + B * C * 4),
    )

    weights = pl.pallas_call(
        functools.partial(_se_kernel, n_hw=n_hw, thw=THW,
                          inv_hw=float(1.0 / HW)),
        out_shape=jax.ShapeDtypeStruct((B, C), jnp.float32),
        in_specs=[
            pl.BlockSpec(memory_space=pl.ANY),
            pl.BlockSpec((Cr, C), lambda: (0, 0)),
            pl.BlockSpec((1, Cr), lambda: (0, 0)),
            pl.BlockSpec((C, Cr), lambda: (0, 0)),
            pl.BlockSpec((1, C), lambda: (0, 0)),
        ],
        out_specs=pl.BlockSpec((B, C), lambda: (0, 0)),
        scratch_shapes=[
            pltpu.VMEM((_DEPTH, THW, B, C), jnp.float32),
            pltpu.SemaphoreType.DMA((_DEPTH,)),
            pltpu.VMEM((B, C), jnp.float32),
        ],
        compiler_params=pltpu.CompilerParams(
            vmem_limit_bytes=vmem_limit,
        ),
        cost_estimate=cost,
    )(xt, w1.astype(jnp.float32), b1r, w2.astype(jnp.float32), b2r)

    return weights.reshape(B, C, 1, 1)


# speedup vs baseline: 1.0241x; 1.0241x over previous
import functools

import jax
import jax.numpy as jnp
from jax import lax
from jax.experimental import pallas as pl
from jax.experimental.pallas import tpu as pltpu


def _se_kernel(x_ref, w1_ref, b1_ref, w2_ref, b2_ref, o_ref, acc, *, inv_hw):
    i = pl.program_id(0)

    @pl.when(i == 0)
    def _():
        acc[...] = jnp.zeros_like(acc)

    acc[...] += jnp.sum(x_ref[...], axis=0)

    @pl.when(i == pl.num_programs(0) - 1)
    def _():
        p = acc[...] * inv_hw
        h = lax.dot_general(p, w1_ref[...], (((1,), (1,)), ((), ())),
                            preferred_element_type=jnp.float32)
        h = jnp.maximum(h + b1_ref[...], 0.0)
        y = lax.dot_general(h, w2_ref[...], (((1,), (1,)), ((), ())),
                            preferred_element_type=jnp.float32)
        o_ref[...] = jax.nn.sigmoid(y + b2_ref[...])


def _pick_thw(HW, B, C, esize, target_bytes):
    best = 1
    for d in range(1, HW + 1):
        if HW % d == 0 and d * B * C * esize <= target_bytes:
            best = d
    return best


def kernel(x_nchw, w1, b1, w2, b2):
    B, C, H, W = x_nchw.shape
    Cr = w1.shape[0]
    HW = H * W
    esize = jnp.dtype(x_nchw.dtype).itemsize

    xt = jnp.transpose(x_nchw, (2, 3, 0, 1)).reshape(HW, B, C)

    THW = _pick_thw(HW, B, C, esize, 7 << 20)
    n_hw = HW // THW

    b1r = b1.reshape(1, Cr).astype(jnp.float32)
    b2r = b2.reshape(1, C).astype(jnp.float32)

    block_bytes = THW * B * C * esize
    vmem_limit = int(min(2 * block_bytes + (8 << 20), 100 << 20))

    cost = pl.CostEstimate(
        flops=int(B * C * HW + 2 * B * C * Cr * 2),
        transcendentals=int(B * C),
        bytes_accessed=int(xt.size * esize + B * C * 4),
    )

    weights = pl.pallas_call(
        functools.partial(_se_kernel, inv_hw=float(1.0 / HW)),
        out_shape=jax.ShapeDtypeStruct((B, C), jnp.float32),
        grid=(n_hw,),
        in_specs=[
            pl.BlockSpec((THW, B, C), lambda i: (i, 0, 0)),
            pl.BlockSpec((Cr, C), lambda i: (0, 0)),
            pl.BlockSpec((1, Cr), lambda i: (0, 0)),
            pl.BlockSpec((C, Cr), lambda i: (0, 0)),
            pl.BlockSpec((1, C), lambda i: (0, 0)),
        ],
        out_specs=pl.BlockSpec((B, C), lambda i: (0, 0)),
        scratch_shapes=[pltpu.VMEM((B, C), jnp.float32)],
        compiler_params=pltpu.CompilerParams(
            dimension_semantics=("arbitrary",),
            vmem_limit_bytes=vmem_limit,
        ),
        cost_estimate=cost,
    )(xt, w1.astype(jnp.float32), b1r, w2.astype(jnp.float32), b2r)

    return weights.reshape(B, C, 1, 1)
